# baseline (speedup 1.0000x reference)
import functools

import jax
import jax.numpy as jnp
from jax import lax
from jax.experimental import pallas as pl
from jax.experimental.pallas import tpu as pltpu
from jax.experimental.pallas import tpu_sc as plsc

BATCH = 16384
D_PER = 64
D_OUT = 2 * D_PER

NUM_CORES = 2
NUM_SUBCORES = 16
NUM_WORKERS = NUM_CORES * NUM_SUBCORES
B_PER_W = BATCH // NUM_WORKERS
CHUNK = 128
N_CHUNKS = B_PER_W // CHUNK

_mesh = plsc.VectorSubcoreMesh(core_axis_name="c", subcore_axis_name="s")


@functools.partial(
    pl.kernel,
    mesh=_mesh,
    compiler_params=pltpu.CompilerParams(use_tc_tiling_on_sc=False),
    out_type=jax.ShapeDtypeStruct((BATCH, D_OUT), jnp.float32),
    scratch_types=[
        pltpu.VMEM((N_CHUNKS, CHUNK), jnp.int32),
        pltpu.VMEM((N_CHUNKS, CHUNK), jnp.int32),
        pltpu.VMEM((B_PER_W, D_PER), jnp.float32),
        pltpu.VMEM((B_PER_W, D_PER), jnp.float32),
        pltpu.SemaphoreType.DMA,
        pltpu.SemaphoreType.DMA,
        pltpu.SemaphoreType.DMA,
        pltpu.SemaphoreType.DMA,
        pltpu.SemaphoreType.DMA,
    ],
)
def _mle_kernel(idx0_hbm, idx1_hbm, w0_hbm, w1_hbm, out_hbm,
                idx0_v, idx1_v, rows0_v, rows1_v,
                g0_sem, g1_sem, g2_sem, g3_sem, o_sem):
    cid = lax.axis_index("c")
    sid = lax.axis_index("s")
    wid = sid * NUM_CORES + cid
    base = wid * B_PER_W

    pltpu.sync_copy(idx0_hbm.at[wid], idx0_v)
    pltpu.sync_copy(idx1_hbm.at[wid], idx1_v)

    gsems = [g0_sem, g1_sem, g2_sem, g3_sem]
    gathers = []
    for c in range(N_CHUNKS):
        rows = pl.ds(c * CHUNK, CHUNK)
        gathers.append((
            pltpu.async_copy(w0_hbm.at[idx0_v.at[c]], rows0_v.at[rows],
                             gsems[c]),
            pltpu.async_copy(w1_hbm.at[idx1_v.at[c]], rows1_v.at[rows],
                             gsems[c]),
        ))

    out_copies = []
    for c in range(N_CHUNKS):
        gathers[c][0].wait()
        gathers[c][1].wait()
        rows = pl.ds(c * CHUNK, CHUNK)
        out_rows = pl.ds(base + c * CHUNK, CHUNK)
        out_copies.append(pltpu.async_copy(
            rows0_v.at[rows], out_hbm.at[out_rows, pl.ds(0, D_PER)], o_sem))
        out_copies.append(pltpu.async_copy(
            rows1_v.at[rows], out_hbm.at[out_rows, pl.ds(D_PER, D_PER)],
            o_sem))
    for cp in out_copies:
        cp.wait()


def kernel(y, W0, W1):
    idx0 = y[:, 0].astype(jnp.int32).reshape(NUM_WORKERS, N_CHUNKS, CHUNK)
    idx1 = y[:, 1].astype(jnp.int32).reshape(NUM_WORKERS, N_CHUNKS, CHUNK)
    return _mle_kernel(idx0, idx1, W0, W1)


# speedup vs baseline: 1.0058x; 1.0058x over previous
import functools

import jax
import jax.numpy as jnp
from jax import lax
from jax.experimental import pallas as pl
from jax.experimental.pallas import tpu as pltpu
from jax.experimental.pallas import tpu_sc as plsc

BATCH = 16384
D_PER = 64
D_OUT = 2 * D_PER

NUM_CORES = 2
NUM_SUBCORES = 16
NUM_WORKERS = NUM_CORES * NUM_SUBCORES
B_PER_W = BATCH // NUM_WORKERS
CHUNK = 256
N_CHUNKS = B_PER_W // CHUNK

_mesh = plsc.VectorSubcoreMesh(core_axis_name="c", subcore_axis_name="s")


@functools.partial(
    pl.kernel,
    mesh=_mesh,
    compiler_params=pltpu.CompilerParams(use_tc_tiling_on_sc=False),
    out_type=jax.ShapeDtypeStruct((BATCH, D_OUT), jnp.float32),
    scratch_types=[
        pltpu.VMEM((N_CHUNKS, CHUNK), jnp.int32),
        pltpu.VMEM((N_CHUNKS, CHUNK), jnp.int32),
        pltpu.VMEM((B_PER_W, D_PER), jnp.float32),
        pltpu.VMEM((B_PER_W, D_PER), jnp.float32),
        pltpu.SemaphoreType.DMA,
        pltpu.SemaphoreType.DMA,
        pltpu.SemaphoreType.DMA,
        pltpu.SemaphoreType.DMA,
        pltpu.SemaphoreType.DMA,
    ],
)
def _mle_kernel(idx0_hbm, idx1_hbm, w0_hbm, w1_hbm, out_hbm,
                idx0_v, idx1_v, rows0_v, rows1_v,
                g0_sem, g1_sem, g2_sem, g3_sem, o_sem):
    cid = lax.axis_index("c")
    sid = lax.axis_index("s")
    wid = sid * NUM_CORES + cid
    base = wid * B_PER_W

    pltpu.sync_copy(idx0_hbm.at[wid], idx0_v)
    pltpu.sync_copy(idx1_hbm.at[wid], idx1_v)

    gsems = [g0_sem, g1_sem, g2_sem, g3_sem]
    gathers = []
    for c in range(N_CHUNKS):
        rows = pl.ds(c * CHUNK, CHUNK)
        gathers.append((
            pltpu.async_copy(w0_hbm.at[idx0_v.at[c]], rows0_v.at[rows],
                             gsems[c]),
            pltpu.async_copy(w1_hbm.at[idx1_v.at[c]], rows1_v.at[rows],
                             gsems[c]),
        ))

    out_copies = []
    for c in range(N_CHUNKS):
        gathers[c][0].wait()
        gathers[c][1].wait()
        rows = pl.ds(c * CHUNK, CHUNK)
        out_rows = pl.ds(base + c * CHUNK, CHUNK)
        out_copies.append(pltpu.async_copy(
            rows0_v.at[rows], out_hbm.at[out_rows, pl.ds(0, D_PER)], o_sem))
        out_copies.append(pltpu.async_copy(
            rows1_v.at[rows], out_hbm.at[out_rows, pl.ds(D_PER, D_PER)],
            o_sem))
    for cp in out_copies:
        cp.wait()


def kernel(y, W0, W1):
    idx0 = y[:, 0].astype(jnp.int32).reshape(NUM_WORKERS, N_CHUNKS, CHUNK)
    idx1 = y[:, 1].astype(jnp.int32).reshape(NUM_WORKERS, N_CHUNKS, CHUNK)
    return _mle_kernel(idx0, idx1, W0, W1)


# speedup vs baseline: 1.0070x; 1.0012x over previous
import functools

import jax
import jax.numpy as jnp
from jax import lax
from jax.experimental import pallas as pl
from jax.experimental.pallas import tpu as pltpu
from jax.experimental.pallas import tpu_sc as plsc

BATCH = 16384
D_PER = 64
D_OUT = 2 * D_PER

NUM_CORES = 2
NUM_SUBCORES = 16
NUM_WORKERS = NUM_CORES * NUM_SUBCORES
B_PER_W = BATCH // NUM_WORKERS
CHUNK = 512
N_CHUNKS = B_PER_W // CHUNK

_mesh = plsc.VectorSubcoreMesh(core_axis_name="c", subcore_axis_name="s")


@functools.partial(
    pl.kernel,
    mesh=_mesh,
    compiler_params=pltpu.CompilerParams(use_tc_tiling_on_sc=False),
    out_type=jax.ShapeDtypeStruct((BATCH, D_OUT), jnp.float32),
    scratch_types=[
        pltpu.VMEM((N_CHUNKS, CHUNK), jnp.int32),
        pltpu.VMEM((N_CHUNKS, CHUNK), jnp.int32),
        pltpu.VMEM((B_PER_W, D_PER), jnp.float32),
        pltpu.VMEM((B_PER_W, D_PER), jnp.float32),
        pltpu.SemaphoreType.DMA,
        pltpu.SemaphoreType.DMA,
        pltpu.SemaphoreType.DMA,
        pltpu.SemaphoreType.DMA,
        pltpu.SemaphoreType.DMA,
    ],
)
def _mle_kernel(idx0_hbm, idx1_hbm, w0_hbm, w1_hbm, out_hbm,
                idx0_v, idx1_v, rows0_v, rows1_v,
                g0_sem, g1_sem, g2_sem, g3_sem, o_sem):
    cid = lax.axis_index("c")
    sid = lax.axis_index("s")
    wid = sid * NUM_CORES + cid
    base = wid * B_PER_W

    pltpu.sync_copy(idx0_hbm.at[wid], idx0_v)
    pltpu.sync_copy(idx1_hbm.at[wid], idx1_v)

    gsems = [g0_sem, g1_sem, g2_sem, g3_sem]
    gathers = []
    for c in range(N_CHUNKS):
        rows = pl.ds(c * CHUNK, CHUNK)
        gathers.append((
            pltpu.async_copy(w0_hbm.at[idx0_v.at[c]], rows0_v.at[rows],
                             gsems[c]),
            pltpu.async_copy(w1_hbm.at[idx1_v.at[c]], rows1_v.at[rows],
                             gsems[c]),
        ))

    out_copies = []
    for c in range(N_CHUNKS):
        gathers[c][0].wait()
        gathers[c][1].wait()
        rows = pl.ds(c * CHUNK, CHUNK)
        out_rows = pl.ds(base + c * CHUNK, CHUNK)
        out_copies.append(pltpu.async_copy(
            rows0_v.at[rows], out_hbm.at[out_rows, pl.ds(0, D_PER)], o_sem))
        out_copies.append(pltpu.async_copy(
            rows1_v.at[rows], out_hbm.at[out_rows, pl.ds(D_PER, D_PER)],
            o_sem))
    for cp in out_copies:
        cp.wait()


def kernel(y, W0, W1):
    idx0 = y[:, 0].astype(jnp.int32).reshape(NUM_WORKERS, N_CHUNKS, CHUNK)
    idx1 = y[:, 1].astype(jnp.int32).reshape(NUM_WORKERS, N_CHUNKS, CHUNK)
    return _mle_kernel(idx0, idx1, W0, W1)
